# baseline (speedup 1.0000x reference)
import functools

import jax
import jax.numpy as jnp
from jax.experimental import pallas as pl
from jax.experimental.pallas import tpu as pltpu

B = 4096
D_MODEL = 512
NUM_TILES = 16
TOP_K = 4
D_FF = 1024
BLK = 512
NBLK = B // BLK


def _routing_body(op_ref, a_ref, b_ref, c_ref, emb_ref, wp_ref, bp_ref,
                  keys_ref, x_ref, xb_ref, dgt_ref, idxt_ref, gatest_ref,
                  cnt_ref, aux_ref, psum_acc, cnt_acc):
    i = pl.program_id(0)

    op_col = op_ref[0]
    a_col = a_ref[0]
    b_col = b_ref[0]
    c_col = c_ref[0]

    iota8 = jax.lax.broadcasted_iota(jnp.int32, (BLK, 8), 1)
    onehot_op = (op_col == iota8).astype(jnp.float32)
    op_emb = jnp.dot(onehot_op, emb_ref[...],
                     preferred_element_type=jnp.float32)
    a_bits = ((a_col >> iota8) & 1).astype(jnp.float32)
    b_bits = ((b_col >> iota8) & 1).astype(jnp.float32)
    zeros7 = jnp.zeros((BLK, 7), jnp.float32)
    feats = jnp.concatenate(
        [op_emb, a_bits, b_bits, c_col.astype(jnp.float32), zeros7], axis=1)

    x = jnp.dot(feats, wp_ref[...],
                preferred_element_type=jnp.float32) + bp_ref[...]
    x_ref[...] = x
    xb_ref[...] = x.astype(jnp.bfloat16)

    logits_t = jax.lax.dot_general(
        keys_ref[...], x, (((1,), (1,)), ((), ())),
        preferred_element_type=jnp.float32)

    iota_e = jax.lax.broadcasted_iota(jnp.int32, (NUM_TILES, BLK), 0)

    cur = logits_t
    vals_rows = []
    idx_rows = []
    for _ in range(TOP_K):
        m = jnp.max(cur, axis=0, keepdims=True)
        is_max = cur == m
        am = jnp.min(jnp.where(is_max, iota_e, NUM_TILES),
                     axis=0, keepdims=True)
        vals_rows.append(m)
        idx_rows.append(am)
        cur = jnp.where(iota_e == am, -jnp.inf, cur)

    vcat = jnp.concatenate(vals_rows, axis=0)
    ecat = jnp.exp(vcat - vals_rows[0])
    gates_t = ecat / jnp.sum(ecat, axis=0, keepdims=True)

    dgt = jnp.zeros((NUM_TILES, BLK), jnp.float32)
    for k in range(TOP_K):
        dgt = dgt + jnp.where(iota_e == idx_rows[k],
                              gates_t[k:k + 1, :], 0.0)
    dgt_ref[...] = dgt
    idxt_ref[...] = jnp.concatenate(idx_rows, axis=0)
    gatest_ref[...] = gates_t

    pe = jnp.exp(logits_t - jnp.max(logits_t, axis=0, keepdims=True))
    probs_t = pe / jnp.sum(pe, axis=0, keepdims=True)
    psum = jnp.sum(probs_t, axis=1, keepdims=True)
    cnt = jnp.sum((dgt > 0.0).astype(jnp.float32), axis=1, keepdims=True)

    @pl.when(i == 0)
    def _init():
        psum_acc[...] = jnp.zeros_like(psum_acc)
        cnt_acc[...] = jnp.zeros_like(cnt_acc)

    psum_acc[...] += jnp.broadcast_to(psum, psum_acc.shape)
    cnt_acc[...] += jnp.broadcast_to(cnt, cnt_acc.shape)

    @pl.when(i == NBLK - 1)
    def _fin():
        prod = psum_acc[:, 0:1] * cnt_acc[:, 0:1]
        s = jnp.sum(prod, keepdims=True)
        aux_ref[...] = s * (NUM_TILES / (B * float(B)))
        cnt_ref[...] = cnt_acc[:, 0:1].astype(jnp.int32)


BLKP = 128
NBLOCKS = (B * TOP_K) // BLKP + NUM_TILES
PADDED = NBLOCKS * BLKP


def _gffn_body(se_ref, x_ref, w1_ref, b1_ref, w2_ref, b2_ref, gate_ref,
               y_ref):
    x = x_ref[...]
    h = jnp.dot(x, w1_ref[0], preferred_element_type=jnp.float32)
    h = jax.nn.gelu(h + b1_ref[0])
    y = jnp.dot(h.astype(jnp.bfloat16), w2_ref[0],
                preferred_element_type=jnp.float32)
    y = y + b2_ref[0]

    ri = jax.lax.broadcasted_iota(jnp.int32, (BLKP, BLKP), 0)
    ci = jax.lax.broadcasted_iota(jnp.int32, (BLKP, BLKP), 1)
    eye = (ri == ci).astype(jnp.float32)
    g_col = jax.lax.dot_general(
        eye, gate_ref[0], (((1,), (1,)), ((), ())),
        preferred_element_type=jnp.float32)

    y_ref[...] = (y * g_col).astype(jnp.bfloat16)


def _heads_body(out_ref, wr_ref, br_ref, wf_ref, bf_ref, res_ref, flg_ref):
    out = out_ref[...]
    res_ref[...] = jax.nn.sigmoid(
        jnp.dot(out, wr_ref[...], preferred_element_type=jnp.float32)
        + br_ref[...])
    flg_ref[...] = jax.nn.sigmoid(
        jnp.dot(out, wf_ref[...], preferred_element_type=jnp.float32)
        + bf_ref[...])


def kernel(op_idx, a, b, c, op_embed, Wp, bp, tile_keys, W1, b1, W2, b2,
           Wr, br, Wf, bf):
    f32 = jnp.float32
    op_r = op_idx.astype(jnp.int32).reshape(NBLK, BLK, 1)
    a_r = a.astype(jnp.int32).reshape(NBLK, BLK, 1)
    b_r = b.astype(jnp.int32).reshape(NBLK, BLK, 1)
    c_r = c.astype(jnp.int32).reshape(NBLK, BLK, 1)
    wp_pad = jnp.zeros((40, D_MODEL), f32).at[:33].set(Wp)

    col_spec = pl.BlockSpec((1, BLK, 1), lambda i: (i, 0, 0))
    x_out, xb_out, dgt, idx_t, gates_t, cnt, aux = pl.pallas_call(
        _routing_body,
        grid=(NBLK,),
        in_specs=[
            col_spec, col_spec, col_spec, col_spec,
            pl.BlockSpec((8, 16), lambda i: (0, 0)),
            pl.BlockSpec((40, D_MODEL), lambda i: (0, 0)),
            pl.BlockSpec((1, D_MODEL), lambda i: (0, 0)),
            pl.BlockSpec((NUM_TILES, D_MODEL), lambda i: (0, 0)),
        ],
        out_specs=[
            pl.BlockSpec((BLK, D_MODEL), lambda i: (i, 0)),
            pl.BlockSpec((BLK, D_MODEL), lambda i: (i, 0)),
            pl.BlockSpec((NUM_TILES, BLK), lambda i: (0, i)),
            pl.BlockSpec((TOP_K, BLK), lambda i: (0, i)),
            pl.BlockSpec((TOP_K, BLK), lambda i: (0, i)),
            pl.BlockSpec((NUM_TILES, 1), lambda i: (0, 0)),
            pl.BlockSpec((1, 1), lambda i: (0, 0)),
        ],
        out_shape=[
            jax.ShapeDtypeStruct((B, D_MODEL), f32),
            jax.ShapeDtypeStruct((B, D_MODEL), jnp.bfloat16),
            jax.ShapeDtypeStruct((NUM_TILES, B), f32),
            jax.ShapeDtypeStruct((TOP_K, B), jnp.int32),
            jax.ShapeDtypeStruct((TOP_K, B), f32),
            jax.ShapeDtypeStruct((NUM_TILES, 1), jnp.int32),
            jax.ShapeDtypeStruct((1, 1), f32),
        ],
        scratch_shapes=[
            pltpu.VMEM((NUM_TILES, 128), f32),
            pltpu.VMEM((NUM_TILES, 128), f32),
        ],
    )(op_r, a_r, b_r, c_r, op_embed, wp_pad, bp.reshape(1, D_MODEL),
      tile_keys)

    e_flat = idx_t.reshape(-1)
    order = jnp.argsort(e_flat, stable=True)
    sorted_e = e_flat[order]
    counts = cnt.reshape(-1)
    ccap = ((counts + BLKP - 1) // BLKP) * BLKP
    cum_cap = jnp.cumsum(ccap)
    pad_off = cum_cap - ccap
    cum_cnt_excl = jnp.cumsum(counts) - counts
    j = jnp.arange(B * TOP_K, dtype=jnp.int32)
    dest = pad_off[sorted_e] + j - cum_cnt_excl[sorted_e]
    token_sorted = jnp.zeros((PADDED,), jnp.int32).at[dest].set(
        (order % B).astype(jnp.int32))
    gate_sorted = jnp.zeros((PADDED,), f32).at[dest].set(
        gates_t.reshape(-1)[order])
    block_expert = jnp.minimum(
        jnp.searchsorted(cum_cap, jnp.arange(NBLOCKS, dtype=jnp.int32) * BLKP,
                         side="right"),
        NUM_TILES - 1).astype(jnp.int32)

    x_sorted = xb_out[token_sorted]

    w1b = W1.astype(jnp.bfloat16)
    w2b = W2.astype(jnp.bfloat16)
    y_sorted = pl.pallas_call(
        _gffn_body,
        grid_spec=pltpu.PrefetchScalarGridSpec(
            num_scalar_prefetch=1,
            grid=(NBLOCKS,),
            in_specs=[
                pl.BlockSpec((BLKP, D_MODEL), lambda b, se: (b, 0)),
                pl.BlockSpec((1, D_MODEL, D_FF), lambda b, se: (se[b], 0, 0)),
                pl.BlockSpec((1, 1, D_FF), lambda b, se: (se[b], 0, 0)),
                pl.BlockSpec((1, D_FF, D_MODEL), lambda b, se: (se[b], 0, 0)),
                pl.BlockSpec((1, 1, D_MODEL), lambda b, se: (se[b], 0, 0)),
                pl.BlockSpec((1, 1, BLKP), lambda b, se: (b, 0, 0)),
            ],
            out_specs=pl.BlockSpec((BLKP, D_MODEL), lambda b, se: (b, 0)),
        ),
        out_shape=jax.ShapeDtypeStruct((PADDED, D_MODEL), jnp.bfloat16),
    )(block_expert, x_sorted, w1b, b1.reshape(NUM_TILES, 1, D_FF), w2b,
      b2.reshape(NUM_TILES, 1, D_MODEL),
      gate_sorted.reshape(NBLOCKS, 1, BLKP))

    out = jnp.zeros((B, D_MODEL), f32).at[token_sorted].add(
        y_sorted.astype(f32))

    result, flags = pl.pallas_call(
        _heads_body,
        grid=(NBLK,),
        in_specs=[
            pl.BlockSpec((BLK, D_MODEL), lambda i: (i, 0)),
            pl.BlockSpec((D_MODEL, 8), lambda i: (0, 0)),
            pl.BlockSpec((1, 8), lambda i: (0, 0)),
            pl.BlockSpec((D_MODEL, 2), lambda i: (0, 0)),
            pl.BlockSpec((1, 2), lambda i: (0, 0)),
        ],
        out_specs=[
            pl.BlockSpec((BLK, 8), lambda i: (i, 0)),
            pl.BlockSpec((BLK, 2), lambda i: (i, 0)),
        ],
        out_shape=[
            jax.ShapeDtypeStruct((B, 8), f32),
            jax.ShapeDtypeStruct((B, 2), f32),
        ],
    )(out, Wr, br.reshape(1, 8), Wf, bf.reshape(1, 2))

    idx = idx_t.T
    return result, flags, idx, aux.reshape(())


# speedup vs baseline: 1.1372x; 1.1372x over previous
import functools

import jax
import jax.numpy as jnp
from jax import lax
from jax.experimental import pallas as pl
from jax.experimental.pallas import tpu as pltpu
from jax.experimental.pallas import tpu_sc as plsc

B = 4096
D_MODEL = 512
NUM_TILES = 16
TOP_K = 4
D_FF = 1024
BLK = 512
NBLK = B // BLK


def _routing_body(op_ref, a_ref, b_ref, c_ref, emb_ref, wp_ref, bp_ref,
                  keys_ref, x_ref, xb_ref, dgt_ref, idxt_ref, gatest_ref,
                  rankt_ref, cnt_ref, aux_ref, psum_acc, cnt_acc):
    i = pl.program_id(0)

    op_col = op_ref[0]
    a_col = a_ref[0]
    b_col = b_ref[0]
    c_col = c_ref[0]

    iota8 = jax.lax.broadcasted_iota(jnp.int32, (BLK, 8), 1)
    onehot_op = (op_col == iota8).astype(jnp.float32)
    op_emb = jnp.dot(onehot_op, emb_ref[...],
                     preferred_element_type=jnp.float32)
    a_bits = ((a_col >> iota8) & 1).astype(jnp.float32)
    b_bits = ((b_col >> iota8) & 1).astype(jnp.float32)
    zeros7 = jnp.zeros((BLK, 7), jnp.float32)
    feats = jnp.concatenate(
        [op_emb, a_bits, b_bits, c_col.astype(jnp.float32), zeros7], axis=1)

    x = jnp.dot(feats, wp_ref[...],
                preferred_element_type=jnp.float32) + bp_ref[...]
    x_ref[...] = x
    xb_ref[...] = x.astype(jnp.bfloat16)

    logits_t = jax.lax.dot_general(
        keys_ref[...], x, (((1,), (1,)), ((), ())),
        preferred_element_type=jnp.float32)

    iota_e = jax.lax.broadcasted_iota(jnp.int32, (NUM_TILES, BLK), 0)

    cur = logits_t
    vals_rows = []
    idx_rows = []
    for _ in range(TOP_K):
        m = jnp.max(cur, axis=0, keepdims=True)
        is_max = cur == m
        am = jnp.min(jnp.where(is_max, iota_e, NUM_TILES),
                     axis=0, keepdims=True)
        vals_rows.append(m)
        idx_rows.append(am)
        cur = jnp.where(iota_e == am, -jnp.inf, cur)

    vcat = jnp.concatenate(vals_rows, axis=0)
    ecat = jnp.exp(vcat - vals_rows[0])
    gates_t = ecat / jnp.sum(ecat, axis=0, keepdims=True)

    dgt = jnp.zeros((NUM_TILES, BLK), jnp.float32)
    for k in range(TOP_K):
        dgt = dgt + jnp.where(iota_e == idx_rows[k],
                              gates_t[k:k + 1, :], 0.0)
    dgt_ref[...] = dgt
    idxt_ref[...] = jnp.concatenate(idx_rows, axis=0)
    gatest_ref[...] = gates_t

    pe = jnp.exp(logits_t - jnp.max(logits_t, axis=0, keepdims=True))
    probs_t = pe / jnp.sum(pe, axis=0, keepdims=True)
    psum = jnp.sum(probs_t, axis=1, keepdims=True)
    cnt = jnp.sum((dgt > 0.0).astype(jnp.float32), axis=1, keepdims=True)

    @pl.when(i == 0)
    def _init():
        psum_acc[...] = jnp.zeros_like(psum_acc)
        cnt_acc[...] = jnp.zeros_like(cnt_acc)

    oh = [(iota_e == idx_rows[k]).astype(jnp.float32) for k in range(TOP_K)]
    ri = jax.lax.broadcasted_iota(jnp.int32, (BLK, BLK), 0)
    ci = jax.lax.broadcasted_iota(jnp.int32, (BLK, BLK), 1)
    tri = (ri < ci).astype(jnp.float32)
    oh_sum = (dgt > 0.0).astype(jnp.float32)
    c_excl = jax.lax.dot_general(
        oh_sum, tri, (((1,), (0,)), ((), ())),
        preferred_element_type=jnp.float32)
    carry = cnt_acc[:, 0:1]
    rank_rows = []
    before = jnp.zeros((NUM_TILES, BLK), jnp.float32)
    for k in range(TOP_K):
        rk = jnp.sum(oh[k] * (carry + c_excl + before),
                     axis=0, keepdims=True)
        rank_rows.append(rk)
        before = before + oh[k]
    rankt_ref[...] = jnp.concatenate(rank_rows, axis=0).astype(jnp.int32)

    psum_acc[...] += jnp.broadcast_to(psum, psum_acc.shape)
    cnt_acc[...] += jnp.broadcast_to(cnt, cnt_acc.shape)

    @pl.when(i == NBLK - 1)
    def _fin():
        prod = psum_acc[:, 0:1] * cnt_acc[:, 0:1]
        s = jnp.sum(prod, keepdims=True)
        aux_ref[...] = s * (NUM_TILES / (B * float(B)))
        cnt_ref[...] = cnt_acc[:, 0:1].astype(jnp.int32)


BLKP = 128
NBLOCKS = (B * TOP_K) // BLKP + NUM_TILES
PADDED = NBLOCKS * BLKP
PADX = PADDED + NUM_TILES * BLKP
NPAIR = B * TOP_K


def _meta_body(idxt_ref, rankt_ref, cnt_ref, destt_ref, dpad_ref):
    counts = cnt_ref[...]
    ccap = ((counts + (BLKP - 1)) // BLKP) * BLKP
    ccap_f = ccap.astype(jnp.float32)
    li = jax.lax.broadcasted_iota(jnp.int32, (NUM_TILES, NUM_TILES), 0)
    lj = jax.lax.broadcasted_iota(jnp.int32, (NUM_TILES, NUM_TILES), 1)
    lower = (li >= lj).astype(jnp.float32)
    cum = jnp.dot(lower, ccap_f, preferred_element_type=jnp.float32)
    pad_off = cum - ccap_f

    iota_e = jax.lax.broadcasted_iota(jnp.int32, (NUM_TILES, B), 0)
    idxt = idxt_ref[...]
    rankt = rankt_ref[...]
    rows = []
    for k in range(TOP_K):
        ohk = (iota_e == idxt[k:k + 1, :]).astype(jnp.float32)
        po = jax.lax.dot_general(
            pad_off, ohk, (((0,), (0,)), ((), ())),
            preferred_element_type=jnp.float32)
        rows.append(rankt[k:k + 1, :] + po.astype(jnp.int32))
    destt_ref[...] = jnp.concatenate(rows, axis=0)

    ij = jax.lax.broadcasted_iota(jnp.int32, (NUM_TILES, BLKP), 1)
    ie = jax.lax.broadcasted_iota(jnp.int32, (NUM_TILES, BLKP), 0)
    base = (pad_off.astype(jnp.int32) + counts)
    valid = (counts + ij) < ccap
    dpad_ref[...] = jnp.where(valid, base + ij,
                              PADDED + ie * BLKP + ij)


def _make_sc_kernels():
    mesh = plsc.VectorSubcoreMesh(core_axis_name="c", subcore_axis_name="s")
    f32 = jnp.float32
    i32 = jnp.int32

    @functools.partial(
        pl.kernel, mesh=mesh,
        out_type=[
            jax.ShapeDtypeStruct((PADX, D_MODEL // 2), i32),
            jax.ShapeDtypeStruct((PADX,), f32),
        ],
        scratch_types=[
            pltpu.VMEM((8, 64), i32),
            pltpu.VMEM((64, D_MODEL // 2), i32),
            pltpu.VMEM((64,), f32),
        ],
    )
    def scatter_build(dest_hbm, gates_hbm, xb_hbm, zrow_hbm, zg_hbm,
                      xs_hbm, gs_hbm, idx_v, row_v, gate_v):
        wid = lax.axis_index("s") * 2 + lax.axis_index("c")
        base_p = pl.multiple_of(wid * (NPAIR // 32), 512)
        t0 = pl.multiple_of(base_p & (B - 1), 512)
        for j in range(8):
            off = pl.multiple_of(base_p + j * 64, 64)
            pltpu.sync_copy(dest_hbm.at[pl.ds(off, 64)], idx_v.at[j])
            pltpu.sync_copy(
                xb_hbm.at[pl.ds(pl.multiple_of(t0 + j * 64, 64), 64)], row_v)
            pltpu.sync_copy(gates_hbm.at[pl.ds(off, 64)], gate_v)
            pltpu.sync_copy(row_v, xs_hbm.at[idx_v.at[j]])
            pltpu.sync_copy(gate_v, gs_hbm.at[idx_v.at[j]])
        pltpu.sync_copy(zrow_hbm, row_v)
        pltpu.sync_copy(zg_hbm, gate_v)
        pltpu.sync_copy(
            dest_hbm.at[pl.ds(pl.multiple_of(NPAIR + wid * 64, 64), 64)],
            idx_v.at[0])
        pltpu.sync_copy(row_v, xs_hbm.at[idx_v.at[0]])
        pltpu.sync_copy(gate_v, gs_hbm.at[idx_v.at[0]])

    @functools.partial(
        pl.kernel, mesh=mesh,
        out_type=jax.ShapeDtypeStruct((B, D_MODEL), f32),
        scratch_types=[
            pltpu.VMEM((4, 32), i32),
            pltpu.VMEM((4, 32, D_MODEL), f32),
            pltpu.VMEM((32, D_MODEL), f32),
        ],
    )
    def combine(dest_hbm, y_hbm, out_hbm, idx_v, ybuf, obuf):
        wid = lax.axis_index("s") * 2 + lax.axis_index("c")
        for ch in range(4):
            t0 = wid * 128 + ch * 32
            for k in range(TOP_K):
                pltpu.sync_copy(dest_hbm.at[pl.ds(k * B + t0, 32)],
                                idx_v.at[k])
            for k in range(TOP_K):
                pltpu.sync_copy(y_hbm.at[idx_v.at[k]], ybuf.at[k])

            def row_body(r, _):
                for v in range(D_MODEL // 16):
                    s = pl.ds(v * 16, 16)
                    acc = (ybuf[0, r, s] + ybuf[1, r, s]
                           + ybuf[2, r, s] + ybuf[3, r, s])
                    obuf[r, s] = acc
                return 0

            lax.fori_loop(0, 32, row_body, 0)
            pltpu.sync_copy(obuf, out_hbm.at[pl.ds(t0, 32)])

    return scatter_build, combine


_scatter_build, _combine = _make_sc_kernels()


def _gffn_body(se_ref, x_ref, w1_ref, b1_ref, w2_ref, b2_ref, gate_ref,
               y_ref):
    x = x_ref[...]
    h = jnp.dot(x, w1_ref[0], preferred_element_type=jnp.float32)
    h = jax.nn.gelu(h + b1_ref[0])
    y = jnp.dot(h.astype(jnp.bfloat16), w2_ref[0],
                preferred_element_type=jnp.float32)
    y = y + b2_ref[0]

    ri = jax.lax.broadcasted_iota(jnp.int32, (BLKP, BLKP), 0)
    ci = jax.lax.broadcasted_iota(jnp.int32, (BLKP, BLKP), 1)
    eye = (ri == ci).astype(jnp.float32)
    g_col = jax.lax.dot_general(
        eye, gate_ref[0], (((1,), (1,)), ((), ())),
        preferred_element_type=jnp.float32)

    y_ref[...] = y * g_col


def _heads_body(out_ref, wr_ref, br_ref, wf_ref, bf_ref, res_ref, flg_ref):
    out = out_ref[...]
    res_ref[...] = jax.nn.sigmoid(
        jnp.dot(out, wr_ref[...], preferred_element_type=jnp.float32)
        + br_ref[...])
    flg_ref[...] = jax.nn.sigmoid(
        jnp.dot(out, wf_ref[...], preferred_element_type=jnp.float32)
        + bf_ref[...])


def kernel(op_idx, a, b, c, op_embed, Wp, bp, tile_keys, W1, b1, W2, b2,
           Wr, br, Wf, bf):
    f32 = jnp.float32
    op_r = op_idx.astype(jnp.int32).reshape(NBLK, BLK, 1)
    a_r = a.astype(jnp.int32).reshape(NBLK, BLK, 1)
    b_r = b.astype(jnp.int32).reshape(NBLK, BLK, 1)
    c_r = c.astype(jnp.int32).reshape(NBLK, BLK, 1)
    wp_pad = jnp.zeros((40, D_MODEL), f32).at[:33].set(Wp)

    col_spec = pl.BlockSpec((1, BLK, 1), lambda i: (i, 0, 0))
    x_out, xb_out, dgt, idx_t, gates_t, rank_t, cnt, aux = pl.pallas_call(
        _routing_body,
        grid=(NBLK,),
        in_specs=[
            col_spec, col_spec, col_spec, col_spec,
            pl.BlockSpec((8, 16), lambda i: (0, 0)),
            pl.BlockSpec((40, D_MODEL), lambda i: (0, 0)),
            pl.BlockSpec((1, D_MODEL), lambda i: (0, 0)),
            pl.BlockSpec((NUM_TILES, D_MODEL), lambda i: (0, 0)),
        ],
        out_specs=[
            pl.BlockSpec((BLK, D_MODEL), lambda i: (i, 0)),
            pl.BlockSpec((BLK, D_MODEL), lambda i: (i, 0)),
            pl.BlockSpec((NUM_TILES, BLK), lambda i: (0, i)),
            pl.BlockSpec((TOP_K, BLK), lambda i: (0, i)),
            pl.BlockSpec((TOP_K, BLK), lambda i: (0, i)),
            pl.BlockSpec((TOP_K, BLK), lambda i: (0, i)),
            pl.BlockSpec((NUM_TILES, 1), lambda i: (0, 0)),
            pl.BlockSpec((1, 1), lambda i: (0, 0)),
        ],
        out_shape=[
            jax.ShapeDtypeStruct((B, D_MODEL), f32),
            jax.ShapeDtypeStruct((B, D_MODEL), jnp.bfloat16),
            jax.ShapeDtypeStruct((NUM_TILES, B), f32),
            jax.ShapeDtypeStruct((TOP_K, B), jnp.int32),
            jax.ShapeDtypeStruct((TOP_K, B), f32),
            jax.ShapeDtypeStruct((TOP_K, B), jnp.int32),
            jax.ShapeDtypeStruct((NUM_TILES, 1), jnp.int32),
            jax.ShapeDtypeStruct((1, 1), f32),
        ],
        scratch_shapes=[
            pltpu.VMEM((NUM_TILES, 128), f32),
            pltpu.VMEM((NUM_TILES, 128), f32),
        ],
    )(op_r, a_r, b_r, c_r, op_embed, wp_pad, bp.reshape(1, D_MODEL),
      tile_keys)

    dest_t, dpad = pl.pallas_call(
        _meta_body,
        grid=(1,),
        in_specs=[
            pl.BlockSpec((TOP_K, B), lambda i: (0, 0)),
            pl.BlockSpec((TOP_K, B), lambda i: (0, 0)),
            pl.BlockSpec((NUM_TILES, 1), lambda i: (0, 0)),
        ],
        out_specs=[
            pl.BlockSpec((TOP_K, B), lambda i: (0, 0)),
            pl.BlockSpec((NUM_TILES, BLKP), lambda i: (0, 0)),
        ],
        out_shape=[
            jax.ShapeDtypeStruct((TOP_K, B), jnp.int32),
            jax.ShapeDtypeStruct((NUM_TILES, BLKP), jnp.int32),
        ],
    )(idx_t, rank_t, cnt)

    dest_all = jnp.concatenate([dest_t.reshape(-1), dpad.reshape(-1)])
    counts = cnt.reshape(-1)
    ccap = ((counts + BLKP - 1) // BLKP) * BLKP
    cum_cap = jnp.cumsum(ccap)
    block_expert = jnp.minimum(
        jnp.searchsorted(cum_cap, jnp.arange(NBLOCKS, dtype=jnp.int32) * BLKP,
                         side="right"),
        NUM_TILES - 1).astype(jnp.int32)

    xb_i32 = jax.lax.bitcast_convert_type(
        xb_out.reshape(B, D_MODEL // 2, 2), jnp.int32)
    xs_i32, gate_sorted = _scatter_build(
        dest_all, gates_t.reshape(-1), xb_i32,
        jnp.zeros((64, D_MODEL // 2), jnp.int32), jnp.zeros((64,), f32))
    x_sorted = jax.lax.bitcast_convert_type(
        xs_i32, jnp.bfloat16).reshape(PADX, D_MODEL)

    w1b = W1.astype(jnp.bfloat16)
    w2b = W2.astype(jnp.bfloat16)
    y_sorted = pl.pallas_call(
        _gffn_body,
        grid_spec=pltpu.PrefetchScalarGridSpec(
            num_scalar_prefetch=1,
            grid=(NBLOCKS,),
            in_specs=[
                pl.BlockSpec((BLKP, D_MODEL), lambda b, se: (b, 0)),
                pl.BlockSpec((1, D_MODEL, D_FF), lambda b, se: (se[b], 0, 0)),
                pl.BlockSpec((1, 1, D_FF), lambda b, se: (se[b], 0, 0)),
                pl.BlockSpec((1, D_FF, D_MODEL), lambda b, se: (se[b], 0, 0)),
                pl.BlockSpec((1, 1, D_MODEL), lambda b, se: (se[b], 0, 0)),
                pl.BlockSpec((1, 1, BLKP), lambda b, se: (b, 0, 0)),
            ],
            out_specs=pl.BlockSpec((BLKP, D_MODEL), lambda b, se: (b, 0)),
        ),
        out_shape=jax.ShapeDtypeStruct((PADDED, D_MODEL), f32),
    )(block_expert, x_sorted, w1b, b1.reshape(NUM_TILES, 1, D_FF), w2b,
      b2.reshape(NUM_TILES, 1, D_MODEL),
      gate_sorted[:PADDED].reshape(NBLOCKS, 1, BLKP))

    out = _combine(dest_t.reshape(-1), y_sorted)

    result, flags = pl.pallas_call(
        _heads_body,
        grid=(NBLK,),
        in_specs=[
            pl.BlockSpec((BLK, D_MODEL), lambda i: (i, 0)),
            pl.BlockSpec((D_MODEL, 8), lambda i: (0, 0)),
            pl.BlockSpec((1, 8), lambda i: (0, 0)),
            pl.BlockSpec((D_MODEL, 2), lambda i: (0, 0)),
            pl.BlockSpec((1, 2), lambda i: (0, 0)),
        ],
        out_specs=[
            pl.BlockSpec((BLK, 8), lambda i: (i, 0)),
            pl.BlockSpec((BLK, 2), lambda i: (i, 0)),
        ],
        out_shape=[
            jax.ShapeDtypeStruct((B, 8), f32),
            jax.ShapeDtypeStruct((B, 2), f32),
        ],
    )(out, Wr, br.reshape(1, 8), Wf, bf.reshape(1, 2))

    idx = idx_t.T
    return result, flags, idx, aux.reshape(())


# speedup vs baseline: 2.0186x; 1.7750x over previous
import functools

import jax
import jax.numpy as jnp
from jax import lax
from jax.experimental import pallas as pl
from jax.experimental.pallas import tpu as pltpu
from jax.experimental.pallas import tpu_sc as plsc

B = 4096
D_MODEL = 512
NUM_TILES = 16
TOP_K = 4
D_FF = 1024
BLK = 512
NBLK = B // BLK


def _routing_body(op_ref, a_ref, b_ref, c_ref, emb_ref, wp_ref, bp_ref,
                  keys_ref, x_ref, xb_ref, dgt_ref, idxt_ref, gatest_ref,
                  rankt_ref, cnt_ref, aux_ref, psum_acc, cnt_acc):
    i = pl.program_id(0)

    op_col = op_ref[0]
    a_col = a_ref[0]
    b_col = b_ref[0]
    c_col = c_ref[0]

    iota8 = jax.lax.broadcasted_iota(jnp.int32, (BLK, 8), 1)
    onehot_op = (op_col == iota8).astype(jnp.float32)
    op_emb = jnp.dot(onehot_op, emb_ref[...],
                     preferred_element_type=jnp.float32)
    a_bits = ((a_col >> iota8) & 1).astype(jnp.float32)
    b_bits = ((b_col >> iota8) & 1).astype(jnp.float32)
    zeros7 = jnp.zeros((BLK, 7), jnp.float32)
    feats = jnp.concatenate(
        [op_emb, a_bits, b_bits, c_col.astype(jnp.float32), zeros7], axis=1)

    x = jnp.dot(feats, wp_ref[...],
                preferred_element_type=jnp.float32) + bp_ref[...]
    x_ref[...] = x
    xb_ref[...] = x.astype(jnp.bfloat16)

    logits_t = jax.lax.dot_general(
        keys_ref[...], x, (((1,), (1,)), ((), ())),
        preferred_element_type=jnp.float32)

    iota_e = jax.lax.broadcasted_iota(jnp.int32, (NUM_TILES, BLK), 0)

    cur = logits_t
    vals_rows = []
    idx_rows = []
    for _ in range(TOP_K):
        m = jnp.max(cur, axis=0, keepdims=True)
        is_max = cur == m
        am = jnp.min(jnp.where(is_max, iota_e, NUM_TILES),
                     axis=0, keepdims=True)
        vals_rows.append(m)
        idx_rows.append(am)
        cur = jnp.where(iota_e == am, -jnp.inf, cur)

    vcat = jnp.concatenate(vals_rows, axis=0)
    ecat = jnp.exp(vcat - vals_rows[0])
    gates_t = ecat / jnp.sum(ecat, axis=0, keepdims=True)

    dgt = jnp.zeros((NUM_TILES, BLK), jnp.float32)
    for k in range(TOP_K):
        dgt = dgt + jnp.where(iota_e == idx_rows[k],
                              gates_t[k:k + 1, :], 0.0)
    dgt_ref[...] = dgt
    idxt_ref[...] = jnp.concatenate(idx_rows, axis=0)
    gatest_ref[...] = gates_t

    pe = jnp.exp(logits_t - jnp.max(logits_t, axis=0, keepdims=True))
    probs_t = pe / jnp.sum(pe, axis=0, keepdims=True)
    psum = jnp.sum(probs_t, axis=1, keepdims=True)
    cnt = jnp.sum((dgt > 0.0).astype(jnp.float32), axis=1, keepdims=True)

    @pl.when(i == 0)
    def _init():
        psum_acc[...] = jnp.zeros_like(psum_acc)
        cnt_acc[...] = jnp.zeros_like(cnt_acc)

    oh = [(iota_e == idx_rows[k]).astype(jnp.float32) for k in range(TOP_K)]
    ri = jax.lax.broadcasted_iota(jnp.int32, (BLK, BLK), 0)
    ci = jax.lax.broadcasted_iota(jnp.int32, (BLK, BLK), 1)
    tri = (ri < ci).astype(jnp.float32)
    oh_sum = (dgt > 0.0).astype(jnp.float32)
    c_excl = jax.lax.dot_general(
        oh_sum, tri, (((1,), (0,)), ((), ())),
        preferred_element_type=jnp.float32)
    carry = cnt_acc[:, 0:1]
    rank_rows = []
    before = jnp.zeros((NUM_TILES, BLK), jnp.float32)
    for k in range(TOP_K):
        rk = jnp.sum(oh[k] * (carry + c_excl + before),
                     axis=0, keepdims=True)
        rank_rows.append(rk)
        before = before + oh[k]
    rankt_ref[...] = jnp.concatenate(rank_rows, axis=0).astype(jnp.int32)

    psum_acc[...] += jnp.broadcast_to(psum, psum_acc.shape)
    cnt_acc[...] += jnp.broadcast_to(cnt, cnt_acc.shape)

    @pl.when(i == NBLK - 1)
    def _fin():
        prod = psum_acc[:, 0:1] * cnt_acc[:, 0:1]
        s = jnp.sum(prod, keepdims=True)
        aux_ref[...] = s * (NUM_TILES / (B * float(B)))
        cnt_ref[...] = cnt_acc[:, 0:1].astype(jnp.int32)


BLKP = 128
NBLOCKS = (B * TOP_K) // BLKP + NUM_TILES
PADDED = NBLOCKS * BLKP
PADX = PADDED + NUM_TILES * BLKP
NPAIR = B * TOP_K


def _meta_body(idxt_ref, rankt_ref, cnt_ref, destt_ref, dpad_ref):
    counts = cnt_ref[...]
    ccap = ((counts + (BLKP - 1)) // BLKP) * BLKP
    ccap_f = ccap.astype(jnp.float32)
    li = jax.lax.broadcasted_iota(jnp.int32, (NUM_TILES, NUM_TILES), 0)
    lj = jax.lax.broadcasted_iota(jnp.int32, (NUM_TILES, NUM_TILES), 1)
    lower = (li >= lj).astype(jnp.float32)
    cum = jnp.dot(lower, ccap_f, preferred_element_type=jnp.float32)
    pad_off = cum - ccap_f

    iota_e = jax.lax.broadcasted_iota(jnp.int32, (NUM_TILES, B), 0)
    idxt = idxt_ref[...]
    rankt = rankt_ref[...]
    rows = []
    for k in range(TOP_K):
        ohk = (iota_e == idxt[k:k + 1, :]).astype(jnp.float32)
        po = jax.lax.dot_general(
            pad_off, ohk, (((0,), (0,)), ((), ())),
            preferred_element_type=jnp.float32)
        rows.append(rankt[k:k + 1, :] + po.astype(jnp.int32))
    destt_ref[...] = jnp.concatenate(rows, axis=0)

    ij = jax.lax.broadcasted_iota(jnp.int32, (NUM_TILES, BLKP), 1)
    ie = jax.lax.broadcasted_iota(jnp.int32, (NUM_TILES, BLKP), 0)
    base = (pad_off.astype(jnp.int32) + counts)
    valid = (counts + ij) < ccap
    dpad_ref[...] = jnp.where(valid, base + ij,
                              PADDED + ie * BLKP + ij)


def _make_sc_kernels():
    mesh = plsc.VectorSubcoreMesh(core_axis_name="c", subcore_axis_name="s")
    f32 = jnp.float32
    i32 = jnp.int32

    @functools.partial(
        pl.kernel, mesh=mesh,
        out_type=[
            jax.ShapeDtypeStruct((PADX, D_MODEL), f32),
            jax.ShapeDtypeStruct((PADX,), f32),
        ],
        scratch_types=[
            pltpu.VMEM((4, 128), i32),
            pltpu.VMEM((128, D_MODEL), f32),
            pltpu.VMEM((4, 128), f32),
            pltpu.VMEM((64,), i32),
        ],
    )
    def scatter_build(dest_hbm, dpad_hbm, gates_hbm, x_hbm, zrow_hbm,
                      zg_hbm, xs_hbm, gs_hbm, idx_v, row_v, gate_v, pidx_v):
        wid = lax.axis_index("s") * 2 + lax.axis_index("c")
        base_p = pl.multiple_of(wid * (NPAIR // 32), 512)
        t0 = pl.multiple_of(base_p & (B - 1), 512)
        pltpu.sync_copy(dest_hbm.at[pl.ds(wid * 4, 4)], idx_v)
        pltpu.sync_copy(gates_hbm.at[pl.ds(wid * 4, 4)], gate_v)
        pltpu.sync_copy(dpad_hbm.at[wid], pidx_v)
        for ch in range(4):
            pltpu.sync_copy(
                x_hbm.at[pl.ds(pl.multiple_of(t0 + ch * 128, 128), 128)],
                row_v)
            pltpu.sync_copy(row_v, xs_hbm.at[idx_v.at[ch]])
            pltpu.sync_copy(gate_v.at[ch], gs_hbm.at[idx_v.at[ch]])
        pltpu.sync_copy(zrow_hbm, row_v.at[pl.ds(0, 64)])
        pltpu.sync_copy(zg_hbm, gate_v.at[0, pl.ds(0, 64)])
        pltpu.sync_copy(row_v.at[pl.ds(0, 64)], xs_hbm.at[pidx_v])
        pltpu.sync_copy(gate_v.at[0, pl.ds(0, 64)], gs_hbm.at[pidx_v])

    @functools.partial(
        pl.kernel, mesh=mesh,
        out_type=jax.ShapeDtypeStruct((B, D_MODEL), f32),
        scratch_types=[
            pltpu.VMEM((4, 128), i32),
            pltpu.VMEM((4, 32, D_MODEL), f32),
            pltpu.VMEM((32, D_MODEL), f32),
        ],
    )
    def combine(dest_hbm, y_hbm, out_hbm, idx_v, ybuf, obuf):
        wid = lax.axis_index("s") * 2 + lax.axis_index("c")
        pltpu.sync_copy(
            dest_hbm.at[:, pl.ds(pl.multiple_of(wid * 128, 128), 128)],
            idx_v)
        for ch in range(4):
            t0 = pl.multiple_of(wid * 128 + ch * 32, 32)
            for k in range(TOP_K):
                pltpu.sync_copy(y_hbm.at[idx_v.at[k, pl.ds(ch * 32, 32)]],
                                ybuf.at[k])

            def row_body(r, _):
                for v in range(D_MODEL // 16):
                    s = pl.ds(v * 16, 16)
                    acc = (ybuf[0, r, s] + ybuf[1, r, s]
                           + ybuf[2, r, s] + ybuf[3, r, s])
                    obuf[r, s] = acc
                return 0

            lax.fori_loop(0, 32, row_body, 0)
            pltpu.sync_copy(obuf, out_hbm.at[pl.ds(t0, 32)])

    return scatter_build, combine


_scatter_build, _combine = _make_sc_kernels()


def _gffn_body(se_ref, x_ref, w1_ref, b1_ref, w2_ref, b2_ref, gate_ref,
               y_ref):
    x = x_ref[...].astype(jnp.bfloat16)
    h = jnp.dot(x, w1_ref[0], preferred_element_type=jnp.float32)
    h = jax.nn.gelu(h + b1_ref[0])
    y = jnp.dot(h.astype(jnp.bfloat16), w2_ref[0],
                preferred_element_type=jnp.float32)
    y = y + b2_ref[0]

    ri = jax.lax.broadcasted_iota(jnp.int32, (BLKP, BLKP), 0)
    ci = jax.lax.broadcasted_iota(jnp.int32, (BLKP, BLKP), 1)
    eye = (ri == ci).astype(jnp.float32)
    g_col = jax.lax.dot_general(
        eye, gate_ref[0], (((1,), (1,)), ((), ())),
        preferred_element_type=jnp.float32)

    y_ref[...] = y * g_col


def _heads_body(out_ref, wr_ref, br_ref, wf_ref, bf_ref, res_ref, flg_ref):
    out = out_ref[...]
    res_ref[...] = jax.nn.sigmoid(
        jnp.dot(out, wr_ref[...], preferred_element_type=jnp.float32)
        + br_ref[...])
    flg_ref[...] = jax.nn.sigmoid(
        jnp.dot(out, wf_ref[...], preferred_element_type=jnp.float32)
        + bf_ref[...])


def kernel(op_idx, a, b, c, op_embed, Wp, bp, tile_keys, W1, b1, W2, b2,
           Wr, br, Wf, bf):
    f32 = jnp.float32
    op_r = op_idx.astype(jnp.int32).reshape(NBLK, BLK, 1)
    a_r = a.astype(jnp.int32).reshape(NBLK, BLK, 1)
    b_r = b.astype(jnp.int32).reshape(NBLK, BLK, 1)
    c_r = c.astype(jnp.int32).reshape(NBLK, BLK, 1)
    wp_pad = jnp.zeros((40, D_MODEL), f32).at[:33].set(Wp)

    col_spec = pl.BlockSpec((1, BLK, 1), lambda i: (i, 0, 0))
    x_out, xb_out, dgt, idx_t, gates_t, rank_t, cnt, aux = pl.pallas_call(
        _routing_body,
        grid=(NBLK,),
        in_specs=[
            col_spec, col_spec, col_spec, col_spec,
            pl.BlockSpec((8, 16), lambda i: (0, 0)),
            pl.BlockSpec((40, D_MODEL), lambda i: (0, 0)),
            pl.BlockSpec((1, D_MODEL), lambda i: (0, 0)),
            pl.BlockSpec((NUM_TILES, D_MODEL), lambda i: (0, 0)),
        ],
        out_specs=[
            pl.BlockSpec((BLK, D_MODEL), lambda i: (i, 0)),
            pl.BlockSpec((BLK, D_MODEL), lambda i: (i, 0)),
            pl.BlockSpec((NUM_TILES, BLK), lambda i: (0, i)),
            pl.BlockSpec((TOP_K, BLK), lambda i: (0, i)),
            pl.BlockSpec((TOP_K, BLK), lambda i: (0, i)),
            pl.BlockSpec((TOP_K, BLK), lambda i: (0, i)),
            pl.BlockSpec((NUM_TILES, 1), lambda i: (0, 0)),
            pl.BlockSpec((1, 1), lambda i: (0, 0)),
        ],
        out_shape=[
            jax.ShapeDtypeStruct((B, D_MODEL), f32),
            jax.ShapeDtypeStruct((B, D_MODEL), jnp.bfloat16),
            jax.ShapeDtypeStruct((NUM_TILES, B), f32),
            jax.ShapeDtypeStruct((TOP_K, B), jnp.int32),
            jax.ShapeDtypeStruct((TOP_K, B), f32),
            jax.ShapeDtypeStruct((TOP_K, B), jnp.int32),
            jax.ShapeDtypeStruct((NUM_TILES, 1), jnp.int32),
            jax.ShapeDtypeStruct((1, 1), f32),
        ],
        scratch_shapes=[
            pltpu.VMEM((NUM_TILES, 128), f32),
            pltpu.VMEM((NUM_TILES, 128), f32),
        ],
    )(op_r, a_r, b_r, c_r, op_embed, wp_pad, bp.reshape(1, D_MODEL),
      tile_keys)

    dest_t, dpad = pl.pallas_call(
        _meta_body,
        grid=(1,),
        in_specs=[
            pl.BlockSpec((TOP_K, B), lambda i: (0, 0)),
            pl.BlockSpec((TOP_K, B), lambda i: (0, 0)),
            pl.BlockSpec((NUM_TILES, 1), lambda i: (0, 0)),
        ],
        out_specs=[
            pl.BlockSpec((TOP_K, B), lambda i: (0, 0)),
            pl.BlockSpec((NUM_TILES, BLKP), lambda i: (0, 0)),
        ],
        out_shape=[
            jax.ShapeDtypeStruct((TOP_K, B), jnp.int32),
            jax.ShapeDtypeStruct((NUM_TILES, BLKP), jnp.int32),
        ],
    )(idx_t, rank_t, cnt)

    counts = cnt.reshape(-1)
    ccap = ((counts + BLKP - 1) // BLKP) * BLKP
    cum_cap = jnp.cumsum(ccap)
    block_expert = jnp.minimum(
        jnp.searchsorted(cum_cap, jnp.arange(NBLOCKS, dtype=jnp.int32) * BLKP,
                         side="right"),
        NUM_TILES - 1).astype(jnp.int32)

    x_sorted, gate_sorted = _scatter_build(
        dest_t.reshape(128, 128), dpad.reshape(32, 64),
        gates_t.reshape(128, 128), x_out,
        jnp.zeros((64, D_MODEL), f32), jnp.zeros((64,), f32))

    w1b = W1.astype(jnp.bfloat16)
    w2b = W2.astype(jnp.bfloat16)
    y_sorted = pl.pallas_call(
        _gffn_body,
        grid_spec=pltpu.PrefetchScalarGridSpec(
            num_scalar_prefetch=1,
            grid=(NBLOCKS,),
            in_specs=[
                pl.BlockSpec((BLKP, D_MODEL), lambda b, se: (b, 0)),
                pl.BlockSpec((1, D_MODEL, D_FF), lambda b, se: (se[b], 0, 0)),
                pl.BlockSpec((1, 1, D_FF), lambda b, se: (se[b], 0, 0)),
                pl.BlockSpec((1, D_FF, D_MODEL), lambda b, se: (se[b], 0, 0)),
                pl.BlockSpec((1, 1, D_MODEL), lambda b, se: (se[b], 0, 0)),
                pl.BlockSpec((1, 1, BLKP), lambda b, se: (b, 0, 0)),
            ],
            out_specs=pl.BlockSpec((BLKP, D_MODEL), lambda b, se: (b, 0)),
        ),
        out_shape=jax.ShapeDtypeStruct((PADDED, D_MODEL), f32),
    )(block_expert, x_sorted, w1b, b1.reshape(NUM_TILES, 1, D_FF), w2b,
      b2.reshape(NUM_TILES, 1, D_MODEL),
      gate_sorted[:PADDED].reshape(NBLOCKS, 1, BLKP))

    out = _combine(dest_t, y_sorted)

    result, flags = pl.pallas_call(
        _heads_body,
        grid=(NBLK,),
        in_specs=[
            pl.BlockSpec((BLK, D_MODEL), lambda i: (i, 0)),
            pl.BlockSpec((D_MODEL, 8), lambda i: (0, 0)),
            pl.BlockSpec((1, 8), lambda i: (0, 0)),
            pl.BlockSpec((D_MODEL, 2), lambda i: (0, 0)),
            pl.BlockSpec((1, 2), lambda i: (0, 0)),
        ],
        out_specs=[
            pl.BlockSpec((BLK, 8), lambda i: (i, 0)),
            pl.BlockSpec((BLK, 2), lambda i: (i, 0)),
        ],
        out_shape=[
            jax.ShapeDtypeStruct((B, 8), f32),
            jax.ShapeDtypeStruct((B, 2), f32),
        ],
    )(out, Wr, br.reshape(1, 8), Wf, bf.reshape(1, 2))

    idx = idx_t.T
    return result, flags, idx, aux.reshape(())


# speedup vs baseline: 2.0674x; 1.0242x over previous
import functools

import jax
import jax.numpy as jnp
from jax import lax
from jax.experimental import pallas as pl
from jax.experimental.pallas import tpu as pltpu
from jax.experimental.pallas import tpu_sc as plsc

B = 4096
D_MODEL = 512
NUM_TILES = 16
TOP_K = 4
D_FF = 1024
BLK = 512
NBLK = B // BLK


def _routing_body(op_ref, a_ref, b_ref, c_ref, emb_ref, wp_ref, bp_ref,
                  keys_ref, x_ref, xb_ref, dgt_ref, idxt_ref, gatest_ref,
                  rankt_ref, cnt_ref, aux_ref, psum_acc, cnt_acc):
    i = pl.program_id(0)

    op_col = op_ref[0]
    a_col = a_ref[0]
    b_col = b_ref[0]
    c_col = c_ref[0]

    iota8 = jax.lax.broadcasted_iota(jnp.int32, (BLK, 8), 1)
    onehot_op = (op_col == iota8).astype(jnp.float32)
    op_emb = jnp.dot(onehot_op, emb_ref[...],
                     preferred_element_type=jnp.float32)
    a_bits = ((a_col >> iota8) & 1).astype(jnp.float32)
    b_bits = ((b_col >> iota8) & 1).astype(jnp.float32)
    zeros7 = jnp.zeros((BLK, 7), jnp.float32)
    feats = jnp.concatenate(
        [op_emb, a_bits, b_bits, c_col.astype(jnp.float32), zeros7], axis=1)

    x = jnp.dot(feats, wp_ref[...],
                preferred_element_type=jnp.float32) + bp_ref[...]
    x_ref[...] = x
    xb_ref[...] = x.astype(jnp.bfloat16)

    logits_t = jax.lax.dot_general(
        keys_ref[...], x, (((1,), (1,)), ((), ())),
        preferred_element_type=jnp.float32)

    iota_e = jax.lax.broadcasted_iota(jnp.int32, (NUM_TILES, BLK), 0)

    cur = logits_t
    vals_rows = []
    idx_rows = []
    for _ in range(TOP_K):
        m = jnp.max(cur, axis=0, keepdims=True)
        is_max = cur == m
        am = jnp.min(jnp.where(is_max, iota_e, NUM_TILES),
                     axis=0, keepdims=True)
        vals_rows.append(m)
        idx_rows.append(am)
        cur = jnp.where(iota_e == am, -jnp.inf, cur)

    vcat = jnp.concatenate(vals_rows, axis=0)
    ecat = jnp.exp(vcat - vals_rows[0])
    gates_t = ecat / jnp.sum(ecat, axis=0, keepdims=True)

    dgt = jnp.zeros((NUM_TILES, BLK), jnp.float32)
    for k in range(TOP_K):
        dgt = dgt + jnp.where(iota_e == idx_rows[k],
                              gates_t[k:k + 1, :], 0.0)
    dgt_ref[...] = dgt
    idxt_ref[...] = jnp.concatenate(idx_rows, axis=0)
    gatest_ref[...] = gates_t

    pe = jnp.exp(logits_t - jnp.max(logits_t, axis=0, keepdims=True))
    probs_t = pe / jnp.sum(pe, axis=0, keepdims=True)
    psum = jnp.sum(probs_t, axis=1, keepdims=True)
    cnt = jnp.sum((dgt > 0.0).astype(jnp.float32), axis=1, keepdims=True)

    @pl.when(i == 0)
    def _init():
        psum_acc[...] = jnp.zeros_like(psum_acc)
        cnt_acc[...] = jnp.zeros_like(cnt_acc)

    oh = [(iota_e == idx_rows[k]).astype(jnp.float32) for k in range(TOP_K)]
    ri = jax.lax.broadcasted_iota(jnp.int32, (BLK, BLK), 0)
    ci = jax.lax.broadcasted_iota(jnp.int32, (BLK, BLK), 1)
    tri = (ri < ci).astype(jnp.float32)
    oh_sum = (dgt > 0.0).astype(jnp.float32)
    c_excl = jax.lax.dot_general(
        oh_sum, tri, (((1,), (0,)), ((), ())),
        preferred_element_type=jnp.float32)
    carry = cnt_acc[:, 0:1]
    rank_rows = []
    before = jnp.zeros((NUM_TILES, BLK), jnp.float32)
    for k in range(TOP_K):
        rk = jnp.sum(oh[k] * (carry + c_excl + before),
                     axis=0, keepdims=True)
        rank_rows.append(rk)
        before = before + oh[k]
    rankt_ref[...] = jnp.concatenate(rank_rows, axis=0).astype(jnp.int32)

    psum_acc[...] += jnp.broadcast_to(psum, psum_acc.shape)
    cnt_acc[...] += jnp.broadcast_to(cnt, cnt_acc.shape)

    @pl.when(i == NBLK - 1)
    def _fin():
        prod = psum_acc[:, 0:1] * cnt_acc[:, 0:1]
        s = jnp.sum(prod, keepdims=True)
        aux_ref[...] = s * (NUM_TILES / (B * float(B)))
        cnt_ref[...] = cnt_acc[:, 0:1].astype(jnp.int32)


BLKP = 128
NBLOCKS = (B * TOP_K) // BLKP + NUM_TILES
PADDED = NBLOCKS * BLKP
PADX = PADDED + NUM_TILES * BLKP
NPAIR = B * TOP_K


def _meta_body(idxt_ref, rankt_ref, cnt_ref, destt_ref, dpad_ref):
    counts = cnt_ref[...]
    ccap = ((counts + (BLKP - 1)) // BLKP) * BLKP
    ccap_f = ccap.astype(jnp.float32)
    li = jax.lax.broadcasted_iota(jnp.int32, (NUM_TILES, NUM_TILES), 0)
    lj = jax.lax.broadcasted_iota(jnp.int32, (NUM_TILES, NUM_TILES), 1)
    lower = (li >= lj).astype(jnp.float32)
    cum = jnp.dot(lower, ccap_f, preferred_element_type=jnp.float32)
    pad_off = cum - ccap_f

    iota_e = jax.lax.broadcasted_iota(jnp.int32, (NUM_TILES, B), 0)
    idxt = idxt_ref[...]
    rankt = rankt_ref[...]
    rows = []
    for k in range(TOP_K):
        ohk = (iota_e == idxt[k:k + 1, :]).astype(jnp.float32)
        po = jax.lax.dot_general(
            pad_off, ohk, (((0,), (0,)), ((), ())),
            preferred_element_type=jnp.float32)
        rows.append(rankt[k:k + 1, :] + po.astype(jnp.int32))
    destt_ref[...] = jnp.concatenate(rows, axis=0)

    ij = jax.lax.broadcasted_iota(jnp.int32, (NUM_TILES, BLKP), 1)
    ie = jax.lax.broadcasted_iota(jnp.int32, (NUM_TILES, BLKP), 0)
    base = (pad_off.astype(jnp.int32) + counts)
    valid = (counts + ij) < ccap
    dpad_ref[...] = jnp.where(valid, base + ij,
                              PADDED + ie * BLKP + ij)


def _make_sc_kernels():
    mesh = plsc.VectorSubcoreMesh(core_axis_name="c", subcore_axis_name="s")
    f32 = jnp.float32
    i32 = jnp.int32

    @functools.partial(
        pl.kernel, mesh=mesh,
        out_type=[
            jax.ShapeDtypeStruct((PADX, D_MODEL), f32),
            jax.ShapeDtypeStruct((PADX,), f32),
        ],
        scratch_types=[
            pltpu.VMEM((8, 64), i32),
            pltpu.VMEM((64, D_MODEL), f32),
            pltpu.VMEM((64, D_MODEL), f32),
            pltpu.VMEM((8, 64), f32),
            pltpu.VMEM((64,), i32),
            pltpu.SemaphoreType.DMA,
            pltpu.SemaphoreType.DMA,
        ],
    )
    def scatter_build(dest_hbm, dpad_hbm, gates_hbm, x_hbm, zrow_hbm,
                      zg_hbm, xs_hbm, gs_hbm, idx_v, row0_v, row1_v,
                      gate_v, pidx_v, sem0, sem1):
        wid = lax.axis_index("s") * 2 + lax.axis_index("c")
        base_p = pl.multiple_of(wid * (NPAIR // 32), 512)
        t0 = pl.multiple_of(base_p & (B - 1), 512)
        pltpu.sync_copy(dest_hbm.at[pl.ds(wid * 8, 8)], idx_v)
        pltpu.sync_copy(gates_hbm.at[pl.ds(wid * 8, 8)], gate_v)
        pltpu.sync_copy(dpad_hbm.at[wid], pidx_v)
        bufs = (row0_v, row1_v)
        sems = (sem0, sem1)
        h_prev = pltpu.async_copy(x_hbm.at[pl.ds(t0, 64)], row0_v, sem0)
        for ch in range(8):
            h_cur = h_prev
            if ch < 7:
                off = pl.multiple_of(t0 + (ch + 1) * 64, 64)
                h_prev = pltpu.async_copy(x_hbm.at[pl.ds(off, 64)],
                                          bufs[(ch + 1) % 2],
                                          sems[(ch + 1) % 2])
            h_cur.wait()
            pltpu.sync_copy(bufs[ch % 2], xs_hbm.at[idx_v.at[ch]])
            pltpu.sync_copy(gate_v.at[ch], gs_hbm.at[idx_v.at[ch]])
        pltpu.sync_copy(zrow_hbm, row0_v)
        pltpu.sync_copy(zg_hbm, gate_v.at[0])
        pltpu.sync_copy(row0_v, xs_hbm.at[pidx_v])
        pltpu.sync_copy(gate_v.at[0], gs_hbm.at[pidx_v])

    @functools.partial(
        pl.kernel, mesh=mesh,
        out_type=jax.ShapeDtypeStruct((B, D_MODEL), f32),
        scratch_types=[
            pltpu.VMEM((4, 128), i32),
            pltpu.VMEM((4, 32, D_MODEL), f32),
            pltpu.VMEM((32, D_MODEL), f32),
            pltpu.SemaphoreType.DMA,
        ],
    )
    def combine(dest_hbm, y_hbm, out_hbm, idx_v, ybuf, obuf, gsem):
        wid = lax.axis_index("s") * 2 + lax.axis_index("c")
        pltpu.sync_copy(
            dest_hbm.at[:, pl.ds(pl.multiple_of(wid * 128, 128), 128)],
            idx_v)
        for ch in range(4):
            t0 = pl.multiple_of(wid * 128 + ch * 32, 32)
            hs = [pltpu.async_copy(
                      y_hbm.at[idx_v.at[k, pl.ds(ch * 32, 32)]],
                      ybuf.at[k], gsem) for k in range(TOP_K)]
            for h in hs:
                h.wait()

            def row_body(r, _):
                for v in range(D_MODEL // 16):
                    s = pl.ds(v * 16, 16)
                    acc = (ybuf[0, r, s] + ybuf[1, r, s]
                           + ybuf[2, r, s] + ybuf[3, r, s])
                    obuf[r, s] = acc
                return 0

            lax.fori_loop(0, 32, row_body, 0)
            pltpu.sync_copy(obuf, out_hbm.at[pl.ds(t0, 32)])

    return scatter_build, combine


_scatter_build, _combine = _make_sc_kernels()


def _gffn_body(se_ref, x_ref, w1_ref, b1_ref, w2_ref, b2_ref, gate_ref,
               y_ref):
    x = x_ref[...].astype(jnp.bfloat16)
    h = jnp.dot(x, w1_ref[0], preferred_element_type=jnp.float32)
    h = jax.nn.gelu(h + b1_ref[0])
    y = jnp.dot(h.astype(jnp.bfloat16), w2_ref[0],
                preferred_element_type=jnp.float32)
    y = y + b2_ref[0]

    ri = jax.lax.broadcasted_iota(jnp.int32, (BLKP, BLKP), 0)
    ci = jax.lax.broadcasted_iota(jnp.int32, (BLKP, BLKP), 1)
    eye = (ri == ci).astype(jnp.float32)
    g_col = jax.lax.dot_general(
        eye, gate_ref[0], (((1,), (1,)), ((), ())),
        preferred_element_type=jnp.float32)

    y_ref[...] = y * g_col


def _heads_body(out_ref, wr_ref, br_ref, wf_ref, bf_ref, res_ref, flg_ref):
    out = out_ref[...]
    res_ref[...] = jax.nn.sigmoid(
        jnp.dot(out, wr_ref[...], preferred_element_type=jnp.float32)
        + br_ref[...])
    flg_ref[...] = jax.nn.sigmoid(
        jnp.dot(out, wf_ref[...], preferred_element_type=jnp.float32)
        + bf_ref[...])


def kernel(op_idx, a, b, c, op_embed, Wp, bp, tile_keys, W1, b1, W2, b2,
           Wr, br, Wf, bf):
    f32 = jnp.float32
    op_r = op_idx.astype(jnp.int32).reshape(NBLK, BLK, 1)
    a_r = a.astype(jnp.int32).reshape(NBLK, BLK, 1)
    b_r = b.astype(jnp.int32).reshape(NBLK, BLK, 1)
    c_r = c.astype(jnp.int32).reshape(NBLK, BLK, 1)
    wp_pad = jnp.zeros((40, D_MODEL), f32).at[:33].set(Wp)

    col_spec = pl.BlockSpec((1, BLK, 1), lambda i: (i, 0, 0))
    x_out, xb_out, dgt, idx_t, gates_t, rank_t, cnt, aux = pl.pallas_call(
        _routing_body,
        grid=(NBLK,),
        in_specs=[
            col_spec, col_spec, col_spec, col_spec,
            pl.BlockSpec((8, 16), lambda i: (0, 0)),
            pl.BlockSpec((40, D_MODEL), lambda i: (0, 0)),
            pl.BlockSpec((1, D_MODEL), lambda i: (0, 0)),
            pl.BlockSpec((NUM_TILES, D_MODEL), lambda i: (0, 0)),
        ],
        out_specs=[
            pl.BlockSpec((BLK, D_MODEL), lambda i: (i, 0)),
            pl.BlockSpec((BLK, D_MODEL), lambda i: (i, 0)),
            pl.BlockSpec((NUM_TILES, BLK), lambda i: (0, i)),
            pl.BlockSpec((TOP_K, BLK), lambda i: (0, i)),
            pl.BlockSpec((TOP_K, BLK), lambda i: (0, i)),
            pl.BlockSpec((TOP_K, BLK), lambda i: (0, i)),
            pl.BlockSpec((NUM_TILES, 1), lambda i: (0, 0)),
            pl.BlockSpec((1, 1), lambda i: (0, 0)),
        ],
        out_shape=[
            jax.ShapeDtypeStruct((B, D_MODEL), f32),
            jax.ShapeDtypeStruct((B, D_MODEL), jnp.bfloat16),
            jax.ShapeDtypeStruct((NUM_TILES, B), f32),
            jax.ShapeDtypeStruct((TOP_K, B), jnp.int32),
            jax.ShapeDtypeStruct((TOP_K, B), f32),
            jax.ShapeDtypeStruct((TOP_K, B), jnp.int32),
            jax.ShapeDtypeStruct((NUM_TILES, 1), jnp.int32),
            jax.ShapeDtypeStruct((1, 1), f32),
        ],
        scratch_shapes=[
            pltpu.VMEM((NUM_TILES, 128), f32),
            pltpu.VMEM((NUM_TILES, 128), f32),
        ],
    )(op_r, a_r, b_r, c_r, op_embed, wp_pad, bp.reshape(1, D_MODEL),
      tile_keys)

    dest_t, dpad = pl.pallas_call(
        _meta_body,
        grid=(1,),
        in_specs=[
            pl.BlockSpec((TOP_K, B), lambda i: (0, 0)),
            pl.BlockSpec((TOP_K, B), lambda i: (0, 0)),
            pl.BlockSpec((NUM_TILES, 1), lambda i: (0, 0)),
        ],
        out_specs=[
            pl.BlockSpec((TOP_K, B), lambda i: (0, 0)),
            pl.BlockSpec((NUM_TILES, BLKP), lambda i: (0, 0)),
        ],
        out_shape=[
            jax.ShapeDtypeStruct((TOP_K, B), jnp.int32),
            jax.ShapeDtypeStruct((NUM_TILES, BLKP), jnp.int32),
        ],
    )(idx_t, rank_t, cnt)

    counts = cnt.reshape(-1)
    ccap = ((counts + BLKP - 1) // BLKP) * BLKP
    cum_cap = jnp.cumsum(ccap)
    block_expert = jnp.minimum(
        jnp.searchsorted(cum_cap, jnp.arange(NBLOCKS, dtype=jnp.int32) * BLKP,
                         side="right"),
        NUM_TILES - 1).astype(jnp.int32)

    x_sorted, gate_sorted = _scatter_build(
        dest_t.reshape(256, 64), dpad.reshape(32, 64),
        gates_t.reshape(256, 64), x_out,
        jnp.zeros((64, D_MODEL), f32), jnp.zeros((64,), f32))

    w1b = W1.astype(jnp.bfloat16)
    w2b = W2.astype(jnp.bfloat16)
    y_sorted = pl.pallas_call(
        _gffn_body,
        grid_spec=pltpu.PrefetchScalarGridSpec(
            num_scalar_prefetch=1,
            grid=(NBLOCKS,),
            in_specs=[
                pl.BlockSpec((BLKP, D_MODEL), lambda b, se: (b, 0)),
                pl.BlockSpec((1, D_MODEL, D_FF), lambda b, se: (se[b], 0, 0)),
                pl.BlockSpec((1, 1, D_FF), lambda b, se: (se[b], 0, 0)),
                pl.BlockSpec((1, D_FF, D_MODEL), lambda b, se: (se[b], 0, 0)),
                pl.BlockSpec((1, 1, D_MODEL), lambda b, se: (se[b], 0, 0)),
                pl.BlockSpec((1, 1, BLKP), lambda b, se: (b, 0, 0)),
            ],
            out_specs=pl.BlockSpec((BLKP, D_MODEL), lambda b, se: (b, 0)),
        ),
        out_shape=jax.ShapeDtypeStruct((PADDED, D_MODEL), f32),
    )(block_expert, x_sorted, w1b, b1.reshape(NUM_TILES, 1, D_FF), w2b,
      b2.reshape(NUM_TILES, 1, D_MODEL),
      gate_sorted[:PADDED].reshape(NBLOCKS, 1, BLKP))

    out = _combine(dest_t, y_sorted)

    result, flags = pl.pallas_call(
        _heads_body,
        grid=(NBLK,),
        in_specs=[
            pl.BlockSpec((BLK, D_MODEL), lambda i: (i, 0)),
            pl.BlockSpec((D_MODEL, 8), lambda i: (0, 0)),
            pl.BlockSpec((1, 8), lambda i: (0, 0)),
            pl.BlockSpec((D_MODEL, 2), lambda i: (0, 0)),
            pl.BlockSpec((1, 2), lambda i: (0, 0)),
        ],
        out_specs=[
            pl.BlockSpec((BLK, 8), lambda i: (i, 0)),
            pl.BlockSpec((BLK, 2), lambda i: (i, 0)),
        ],
        out_shape=[
            jax.ShapeDtypeStruct((B, 8), f32),
            jax.ShapeDtypeStruct((B, 2), f32),
        ],
    )(out, Wr, br.reshape(1, 8), Wf, bf.reshape(1, 2))

    idx = idx_t.T
    return result, flags, idx, aux.reshape(())
